# Initial kernel scaffold; baseline (speedup 1.0000x reference)
import jax
import jax.numpy as jnp
from jax.experimental import pallas as pl


def kernel(H, noise_u):
    raise NotImplementedError("write your pallas kernel here")


# speedup vs baseline: 12.3805x; 12.3805x over previous
import jax
import jax.numpy as jnp
import numpy as np
from jax.experimental import pallas as pl
from jax.experimental.pallas import tpu as pltpu

_T = 500
_HALO = 8


def _make_wt(nbands: int) -> np.ndarray:
    N = 2 * nbands - 1
    m = np.arange(N, dtype=np.float64)
    k = np.arange(nbands, dtype=np.float64)
    wk = np.full(nbands, 2.0, dtype=np.float64)
    wk[0] = 1.0
    hann = 0.5 * (1.0 - np.cos(2.0 * np.pi * m / N))
    ang = (2.0 * np.pi / N) * np.outer(m - (nbands - 1), k)
    wt = np.cos(ang) * (wk / N)[None, :] * hann[:, None]
    return wt.astype(np.float32)


def _body(wt_ref, hc_ref, hh_ref, nc_ref, nh_ref, out_ref):
    F = 80
    h_full = jnp.concatenate([hh_ref[0, 0], hc_ref[0, 0]], axis=1)
    nz = jnp.concatenate([nh_ref[0, 0], nc_ref[0, 0]], axis=1)
    nz = nz * 2.0 - 1.0
    L = h_full.shape[1]

    fir = jnp.dot(wt_ref[:, :], h_full, preferred_element_type=jnp.float32)
    fir = jnp.pad(fir, ((0, 7), (0, 0)))

    acc = None
    for s in range(8):
        a_s = None
        for q in range(10):
            a = 8 * q + s
            prod = nz[a : a + 1, :] * fir
            term = jnp.pad(prod, ((8 * q, 72 - 8 * q), (0, 0)))
            a_s = term if a_s is None else a_s + term
        if s:
            a_s = jnp.pad(a_s, ((s, 0), (0, 0)))[:592]
        acc = a_s if acc is None else acc + a_s

    out = None
    for p in range(8):
        piece = acc[F * p : F * p + F, _HALO - p : L - p]
        if p == 7:
            piece = jnp.pad(piece, ((0, 48), (0, 0)))
        out = piece if out is None else out + piece
    out_ref[0, 0] = out


def kernel(H, noise_u):
    B, t, nbands = H.shape
    F = noise_u.shape[-1]
    N = 2 * nbands - 1
    T = _T
    nc = t // T
    wt = jnp.asarray(_make_wt(nbands))

    hc = H.reshape(B, nc, T, nbands).transpose(1, 0, 3, 2)
    nzc = noise_u.reshape(B, nc, T, F).transpose(1, 0, 3, 2)
    hp = jnp.pad(H, ((0, 0), (_HALO, 0), (0, 0)))[:, :t]
    hh = hp.reshape(B, nc, T, nbands)[:, :, :_HALO].transpose(1, 0, 3, 2)
    np_ = jnp.pad(noise_u, ((0, 0), (_HALO, 0), (0, 0)))[:, :t]
    nh = np_.reshape(B, nc, T, F)[:, :, :_HALO].transpose(1, 0, 3, 2)

    out4 = pl.pallas_call(
        _body,
        grid=(B, nc),
        in_specs=[
            pl.BlockSpec((N, nbands), lambda b, i: (0, 0)),
            pl.BlockSpec((1, 1, nbands, T), lambda b, i: (i, b, 0, 0)),
            pl.BlockSpec((1, 1, nbands, _HALO), lambda b, i: (i, b, 0, 0)),
            pl.BlockSpec((1, 1, F, T), lambda b, i: (i, b, 0, 0)),
            pl.BlockSpec((1, 1, F, _HALO), lambda b, i: (i, b, 0, 0)),
        ],
        out_specs=pl.BlockSpec((1, 1, F, T), lambda b, i: (b, i, 0, 0)),
        out_shape=jax.ShapeDtypeStruct((B, nc, F, T), jnp.float32),
        compiler_params=pltpu.CompilerParams(
            dimension_semantics=("parallel", "parallel")
        ),
    )(wt, hc, hh, nzc, nh)

    return out4.transpose(0, 1, 3, 2).reshape(B, t * F)
